# baseline (speedup 1.0000x reference)
import functools

import jax
import jax.numpy as jnp
from jax import lax
from jax.experimental import pallas as pl
from jax.experimental.pallas import tpu as pltpu
from jax.experimental.pallas import tpu_sc as plsc

_N_ENT = 10000
_N_ENT_PAD = 10240
_EMB = 128
_HALF = 64
_N_EDGE = 320000
_NT = 16
_EPT = _N_EDGE // _NT
_CHUNK = 128
_NS = 6
_NPROC = 162
_NTOT = _NPROC + _NS - 1
_EPT_PAD = _NTOT * _CHUNK
_ROWS_PER_TILE = _N_ENT_PAD // _NT
_WCHUNK = 128
_NWCHUNK = _ROWS_PER_TILE // _WCHUNK


def _make_edge_kernel():
    mesh = plsc.VectorSubcoreMesh(core_axis_name="c", subcore_axis_name="s")

    @functools.partial(
        pl.kernel,
        out_type=(
            jax.ShapeDtypeStruct((2, _N_ENT_PAD, _HALF), jnp.float32),
            jax.ShapeDtypeStruct((2, _N_ENT_PAD), jnp.float32),
        ),
        mesh=mesh,
        scratch_types=[
            pltpu.VMEM((_NTOT, _CHUNK), jnp.int32),
            pltpu.VMEM((_NS, _CHUNK), jnp.int32),
            pltpu.VMEM((_NS, _CHUNK), jnp.int32),
            pltpu.VMEM((_NS, _CHUNK), jnp.int32),
        ] + [
            pltpu.VMEM((_CHUNK, _HALF), jnp.float32)
            for _ in range(_NS)
        ] + [
            pltpu.VMEM((16, _HALF), jnp.float32),
            pltpu.VMEM((_CHUNK,), jnp.float32),
            pltpu.VMEM((_WCHUNK, _HALF), jnp.float32),
            pltpu.VMEM((_ROWS_PER_TILE,), jnp.float32),
            pltpu.VMEM_SHARED((_N_ENT_PAD, _HALF), jnp.float32),
            pltpu.VMEM_SHARED((_N_ENT_PAD,), jnp.float32),
        ] + [pltpu.SemaphoreType.DMA] * (2 * _NS),
        compiler_params=pltpu.CompilerParams(use_tc_tiling_on_sc=False),
    )
    def edge_kernel(packed_hbm, ent_hbm, remb_hbm,
                    psum_hbm, pcnt_hbm,
                    pk_v, tl_s, hd_s, rl_s, *rest):
        rows = rest[:_NS]
        remb_v, ones_v, stage_v, cnt_v, acc_sum, acc_cnt = rest[_NS:_NS + 6]
        gsems = rest[_NS + 6:_NS + 6 + _NS]
        ssems = rest[_NS + 6 + _NS:]
        c = lax.axis_index("c")
        s = lax.axis_index("s")
        base = s * _ROWS_PER_TILE

        pltpu.sync_copy(remb_hbm.at[c], remb_v)
        pltpu.sync_copy(packed_hbm.at[s], pk_v)

        for i in range(_CHUNK // 16):
            ones_v[pl.ds(i * 16, 16)] = jnp.ones((16,), jnp.float32)

        def _zero_stage(i, _):
            for j in range(_HALF // 16):
                stage_v[i, pl.ds(j * 16, 16)] = jnp.zeros((16,), jnp.float32)
            return 0
        lax.fori_loop(0, _WCHUNK, _zero_stage, 0)

        def _zero_cnt(i, _):
            cnt_v[pl.ds(i * 16, 16)] = jnp.zeros((16,), jnp.float32)
            return 0
        lax.fori_loop(0, _ROWS_PER_TILE // 16, _zero_cnt, 0)

        for i in range(_NWCHUNK):
            pltpu.sync_copy(stage_v, acc_sum.at[pl.ds(base + i * _WCHUNK, _WCHUNK)])
        pltpu.sync_copy(cnt_v, acc_cnt.at[pl.ds(base, _ROWS_PER_TILE)])

        plsc.subcore_barrier()

        def _unpack(kc, j):
            for g in range(_CHUNK // 16):
                sl = pl.ds(g * 16, 16)
                p = pk_v[kc, sl]
                tl_s[j, sl] = p & 0x3FFF
                hd_s[j, sl] = (p >> 14) & 0x3FFF
                rl_s[j, sl] = (p >> 28) & 0xF

        def _gather(j):
            pltpu.async_copy(ent_hbm.at[c].at[tl_s.at[j]], rows[j], gsems[j])

        def _wait_gather(j):
            pltpu.make_async_copy(
                ent_hbm.at[c].at[tl_s.at[j]], rows[j], gsems[j]).wait()

        def _mult(j):
            buf = rows[j]

            def _group(g, _):
                relv = rl_s[j, pl.ds(g * 16, 16)]
                e0 = g * 16
                nj = _HALF // 16
                for l in range(16):
                    r = relv[l]
                    e = e0 + l
                    a = [buf[e, pl.ds(jj * 16, 16)] for jj in range(nj)]
                    b = [remb_v[r, pl.ds(jj * 16, 16)] for jj in range(nj)]
                    for jj in range(nj):
                        buf[e, pl.ds(jj * 16, 16)] = a[jj] * b[jj]
                return 0
            lax.fori_loop(0, _CHUNK // 16, _group, 0)

        def _scatter(j):
            pltpu.async_copy(rows[j], acc_sum.at[hd_s.at[j]], ssems[j], add=True)
            pltpu.async_copy(ones_v, acc_cnt.at[hd_s.at[j]], ssems[j], add=True)

        def _wait_scatter(j):
            pltpu.make_async_copy(rows[j], acc_sum.at[hd_s.at[j]], ssems[j]).wait()
            pltpu.make_async_copy(ones_v, acc_cnt.at[hd_s.at[j]], ssems[j]).wait()

        for j in range(_NS):
            _unpack(j, j)
        for j in range(_NS):
            _gather(j)
        for j in range(_NS):
            _wait_gather(j)
            _mult(j)
            _scatter(j)
            if j > 0:
                _wait_scatter(j - 1)
                _unpack(j - 1 + _NS, j - 1)
                _gather(j - 1)

        def _ring(k, _):
            base_kc = _NS * k + _NS
            for j in range(_NS):
                kc = base_kc + j
                pj = (j - 1) % _NS
                _wait_gather(j)
                _mult(j)
                _scatter(j)
                _wait_scatter(pj)
                _unpack(kc + _NS - 1, pj)
                _gather(pj)
            return 0
        lax.fori_loop(0, (_NPROC - _NS) // _NS, _ring, 0)

        _wait_scatter(_NS - 1)
        for j in range(_NS - 1):
            _wait_gather(j)

        plsc.subcore_barrier()

        for i in range(_NWCHUNK):
            off = base + i * _WCHUNK
            pltpu.sync_copy(acc_sum.at[pl.ds(off, _WCHUNK)], stage_v)
            pltpu.sync_copy(stage_v, psum_hbm.at[c, pl.ds(off, _WCHUNK)])
        pltpu.sync_copy(acc_cnt.at[pl.ds(base, _ROWS_PER_TILE)], cnt_v)
        pltpu.sync_copy(cnt_v, pcnt_hbm.at[c, pl.ds(base, _ROWS_PER_TILE)])

    return edge_kernel


_edge_agg = _make_edge_kernel()


def _finalize_body(ps_ref, pc_ref, out_ref):
    sums = jnp.concatenate([ps_ref[0], ps_ref[1]], axis=1)
    out_ref[...] = sums / jnp.maximum(pc_ref[...], 1.0)


def _finalize(psum, pcnt):
    blk = 2048
    grid = _N_ENT_PAD // blk
    return pl.pallas_call(
        _finalize_body,
        grid=(grid,),
        in_specs=[
            pl.BlockSpec((2, blk, _HALF), lambda i: (0, i, 0)),
            pl.BlockSpec((blk, 1), lambda i: (i, 0)),
        ],
        out_specs=pl.BlockSpec((blk, _EMB), lambda i: (i, 0)),
        out_shape=jax.ShapeDtypeStruct((_N_ENT_PAD, _EMB), jnp.float32),
    )(psum, pcnt[0].reshape(_N_ENT_PAD, 1))


def _intent_vec(row, sub):
    logits = jnp.sum(row * sub, axis=1, keepdims=True)
    m = jnp.max(logits, axis=0, keepdims=True)
    e = jnp.exp(logits - m)
    att = e / jnp.sum(e, axis=0, keepdims=True)
    return jnp.sum(att * sub, axis=0, keepdims=True) / sub.shape[0]


def _user_body(u_ref, im_ref, ent_ref, it_ref, r_ref, out_ref):
    it = it_ref[...]
    r = r_ref[...]
    parts = [
        _intent_vec(it[0:1], r),
        _intent_vec(it[1:2], r[0:4]),
        _intent_vec(it[2:3], r[4:8]),
        _intent_vec(it[3:4], r[8:12]),
        _intent_vec(it[4:5], r[12:16]),
    ]
    all_intent = jnp.concatenate(parts, axis=0)
    new_intent = (all_intent + it) * 0.5

    u = u_ref[...]
    score_ = jax.lax.dot_general(
        u, new_intent, (((1,), (1,)), ((), ())),
        preferred_element_type=jnp.float32)
    sm = jnp.max(score_, axis=1, keepdims=True)
    se = jnp.exp(score_ - sm)
    score = se / jnp.sum(se, axis=1, keepdims=True)

    wvec = jax.lax.dot_general(
        score, new_intent, (((1,), (0,)), ((), ())),
        preferred_element_type=jnp.float32)

    agg = jax.lax.dot_general(
        im_ref[...], ent_ref[...], (((1,), (0,)), ((), ())),
        preferred_element_type=jnp.float32)
    out_ref[...] = agg * (1.0 + wvec)


def _user_agg(user_emb, interact_mat, entity_emb, intent_emb, r_emb):
    n_users = user_emb.shape[0]
    blk = 512
    grid = n_users // blk
    return pl.pallas_call(
        _user_body,
        grid=(grid,),
        in_specs=[
            pl.BlockSpec((blk, _EMB), lambda i: (i, 0)),
            pl.BlockSpec((blk, _N_ENT), lambda i: (i, 0)),
            pl.BlockSpec((_N_ENT, _EMB), lambda i: (0, 0)),
            pl.BlockSpec((5, _EMB), lambda i: (0, 0)),
            pl.BlockSpec((16, _EMB), lambda i: (0, 0)),
        ],
        out_specs=pl.BlockSpec((blk, _EMB), lambda i: (i, 0)),
        out_shape=jax.ShapeDtypeStruct((n_users, _EMB), jnp.float32),
    )(user_emb, interact_mat, entity_emb, intent_emb, r_emb)


def kernel(entity_emb, user_emb, intent_emb, edge_index, edge_type, interact_mat, r_emb):
    npad = _EPT_PAD - _EPT
    head = edge_index[0].astype(jnp.int32)
    tail = edge_index[1].astype(jnp.int32)
    rel = (edge_type.astype(jnp.int32) - 1) & 15
    packed = (tail | (head << 14) | (rel << 28)).reshape(_NT, _EPT)
    packed = jnp.concatenate(
        [packed, jnp.full((_NT, npad), _N_ENT << 14, jnp.int32)], axis=1
    ).reshape(_NT, _NTOT, _CHUNK)

    ent_halves = jnp.stack([entity_emb[:, :_HALF], entity_emb[:, _HALF:]])
    remb_halves = jnp.stack([r_emb[:, :_HALF], r_emb[:, _HALF:]])

    psum, pcnt = _edge_agg(packed, ent_halves, remb_halves)
    entity_agg = _finalize(psum, pcnt)[:_N_ENT]
    user_agg = _user_agg(user_emb, interact_mat, entity_emb, intent_emb, r_emb)
    return entity_agg, user_agg


# speedup vs baseline: 2.2002x; 2.2002x over previous
import functools

import numpy as np

import jax
import jax.numpy as jnp
from jax import lax
from jax.experimental import pallas as pl
from jax.experimental.pallas import tpu as pltpu
from jax.experimental.pallas import tpu_sc as plsc

_N_ENT = 10000
_N_ENT_PAD = 10240
_EMB = 128
_HALF = 64
_HW = _HALF // 2
_N_EDGE = 320000
_NT = 16
_EPT = _N_EDGE // _NT
_CHUNK = 128
_NS = 3
_NPROC = 159
_NTOT = _NPROC + _NS - 1
_EPT_PAD = _NTOT * _CHUNK
_ROWS_PER_TILE = _N_ENT_PAD // _NT
_WCHUNK = 128
_NWCHUNK = _ROWS_PER_TILE // _WCHUNK
_TROWS = _N_ENT // _NT
_TCH = 125

_P = ([2 * i for i in range(16)] + [2 * i + 1 for i in range(16)]
      + [32 + 2 * i for i in range(16)] + [33 + 2 * i for i in range(16)])
_PINV = np.argsort(np.array(_P))


def _make_edge_kernel():
    mesh = plsc.VectorSubcoreMesh(core_axis_name="c", subcore_axis_name="s")

    @functools.partial(
        pl.kernel,
        out_type=(
            jax.ShapeDtypeStruct((2, _N_ENT_PAD, _HALF), jnp.float32),
            jax.ShapeDtypeStruct((2, _N_ENT_PAD), jnp.float32),
        ),
        mesh=mesh,
        scratch_types=[
            pltpu.VMEM((_NTOT, _CHUNK), jnp.int32),
            pltpu.VMEM((_NS, _CHUNK), jnp.int32),
            pltpu.VMEM((_NS, _CHUNK), jnp.int32),
            pltpu.VMEM((_NS, _CHUNK), jnp.int32),
        ] + [
            pltpu.VMEM((_CHUNK, _HW), jnp.int32)
            for _ in range(_NS)
        ] + [
            pltpu.VMEM((_CHUNK, _HALF), jnp.float32)
            for _ in range(_NS)
        ] + [
            pltpu.VMEM((16, _HALF), jnp.float32),
            pltpu.VMEM((_CHUNK,), jnp.float32),
            pltpu.VMEM((_ROWS_PER_TILE,), jnp.float32),
            pltpu.VMEM_SHARED((_N_ENT, _HW), jnp.int32),
            pltpu.VMEM_SHARED((_N_ENT_PAD, _HALF), jnp.float32),
            pltpu.VMEM_SHARED((_N_ENT_PAD,), jnp.float32),
        ] + [pltpu.SemaphoreType.DMA] * (2 * _NS),
        compiler_params=pltpu.CompilerParams(use_tc_tiling_on_sc=False),
    )
    def edge_kernel(packed_hbm, tbl_hbm, remb_hbm,
                    psum_hbm, pcnt_hbm,
                    pk_v, tl_s, hd_s, rl_s, *rest):
        gbufs = rest[:_NS]
        obufs = rest[_NS:2 * _NS]
        remb_v, ones_v, cnt_v, tbl_spm, acc_sum, acc_cnt = rest[2 * _NS:2 * _NS + 6]
        gsems = rest[2 * _NS + 6:2 * _NS + 6 + _NS]
        ssems = rest[2 * _NS + 6 + _NS:]
        c = lax.axis_index("c")
        s = lax.axis_index("s")
        base = s * _ROWS_PER_TILE

        pltpu.sync_copy(remb_hbm.at[c], remb_v)
        pltpu.sync_copy(packed_hbm.at[s], pk_v)

        tstage = gbufs[0].at[pl.ds(0, _TCH)]
        for i in range(_TROWS // _TCH):
            off = s * _TROWS + i * _TCH
            pltpu.sync_copy(tbl_hbm.at[c, pl.ds(off, _TCH)], tstage)
            pltpu.sync_copy(tstage, tbl_spm.at[pl.ds(off, _TCH)])

        for i in range(_CHUNK // 16):
            ones_v[pl.ds(i * 16, 16)] = jnp.ones((16,), jnp.float32)

        zbuf = obufs[0]

        def _zero_stage(i, _):
            for j in range(_HALF // 16):
                zbuf[i, pl.ds(j * 16, 16)] = jnp.zeros((16,), jnp.float32)
            return 0
        lax.fori_loop(0, _WCHUNK, _zero_stage, 0)

        def _zero_cnt(i, _):
            cnt_v[pl.ds(i * 16, 16)] = jnp.zeros((16,), jnp.float32)
            return 0
        lax.fori_loop(0, _ROWS_PER_TILE // 16, _zero_cnt, 0)

        for i in range(_NWCHUNK):
            pltpu.sync_copy(zbuf, acc_sum.at[pl.ds(base + i * _WCHUNK, _WCHUNK)])
        pltpu.sync_copy(cnt_v, acc_cnt.at[pl.ds(base, _ROWS_PER_TILE)])

        plsc.subcore_barrier()

        def _unpack(kc, j):
            for g in range(_CHUNK // 16):
                sl = pl.ds(g * 16, 16)
                p = pk_v[kc, sl]
                tl_s[j, sl] = p & 0x3FFF
                hd_s[j, sl] = (p >> 14) & 0x3FFF
                rl_s[j, sl] = (p >> 28) & 0xF

        def _gather(j):
            pltpu.async_copy(tbl_spm.at[tl_s.at[j]], gbufs[j], gsems[j])

        def _wait_gather(j):
            pltpu.make_async_copy(
                tbl_spm.at[tl_s.at[j]], gbufs[j], gsems[j]).wait()

        def _mult(j):
            gb = gbufs[j]
            ob = obufs[j]
            mask = jnp.int32(-65536)

            def _group(g, _):
                relv = rl_s[j, pl.ds(g * 16, 16)]
                e0 = g * 16
                for l in range(16):
                    r = relv[l]
                    e = e0 + l
                    x0 = gb[e, pl.ds(0, 16)]
                    x1 = gb[e, pl.ds(16, 16)]
                    b = [remb_v[r, pl.ds(jj * 16, 16)] for jj in range(4)]
                    lo0 = lax.bitcast_convert_type(x0 << 16, jnp.float32)
                    hi0 = lax.bitcast_convert_type(x0 & mask, jnp.float32)
                    lo1 = lax.bitcast_convert_type(x1 << 16, jnp.float32)
                    hi1 = lax.bitcast_convert_type(x1 & mask, jnp.float32)
                    ob[e, pl.ds(0, 16)] = lo0 * b[0]
                    ob[e, pl.ds(16, 16)] = hi0 * b[1]
                    ob[e, pl.ds(32, 16)] = lo1 * b[2]
                    ob[e, pl.ds(48, 16)] = hi1 * b[3]
                return 0
            lax.fori_loop(0, _CHUNK // 16, _group, 0)

        def _scatter(j):
            pltpu.async_copy(obufs[j], acc_sum.at[hd_s.at[j]], ssems[j], add=True)
            pltpu.async_copy(ones_v, acc_cnt.at[hd_s.at[j]], ssems[j], add=True)

        def _wait_scatter(j):
            pltpu.make_async_copy(obufs[j], acc_sum.at[hd_s.at[j]], ssems[j]).wait()
            pltpu.make_async_copy(ones_v, acc_cnt.at[hd_s.at[j]], ssems[j]).wait()

        for j in range(_NS):
            _unpack(j, j)
        for j in range(_NS):
            _gather(j)
        for j in range(_NS):
            _wait_gather(j)
            _mult(j)
            _scatter(j)
            if j > 0:
                _wait_scatter(j - 1)
                _unpack(j - 1 + _NS, j - 1)
                _gather(j - 1)

        def _ring(k, _):
            base_kc = _NS * k + _NS
            for j in range(_NS):
                kc = base_kc + j
                pj = (j - 1) % _NS
                _wait_gather(j)
                _mult(j)
                _scatter(j)
                _wait_scatter(pj)
                _unpack(kc + _NS - 1, pj)
                _gather(pj)
            return 0
        lax.fori_loop(0, (_NPROC - _NS) // _NS, _ring, 0)

        _wait_scatter(_NS - 1)
        for j in range(_NS - 1):
            _wait_gather(j)

        plsc.subcore_barrier()

        for i in range(_NWCHUNK):
            off = base + i * _WCHUNK
            pltpu.sync_copy(acc_sum.at[pl.ds(off, _WCHUNK)], zbuf)
            pltpu.sync_copy(zbuf, psum_hbm.at[c, pl.ds(off, _WCHUNK)])
        pltpu.sync_copy(acc_cnt.at[pl.ds(base, _ROWS_PER_TILE)], cnt_v)
        pltpu.sync_copy(cnt_v, pcnt_hbm.at[c, pl.ds(base, _ROWS_PER_TILE)])

    return edge_kernel


_edge_agg = _make_edge_kernel()


def _finalize_body(ps_ref, pc_ref, out_ref):
    sums = jnp.concatenate([ps_ref[0], ps_ref[1]], axis=1)
    out_ref[...] = sums / jnp.maximum(pc_ref[...], 1.0)


def _finalize(psum, pcnt):
    blk = 2048
    grid = _N_ENT_PAD // blk
    return pl.pallas_call(
        _finalize_body,
        grid=(grid,),
        in_specs=[
            pl.BlockSpec((2, blk, _HALF), lambda i: (0, i, 0)),
            pl.BlockSpec((blk, 1), lambda i: (i, 0)),
        ],
        out_specs=pl.BlockSpec((blk, _EMB), lambda i: (i, 0)),
        out_shape=jax.ShapeDtypeStruct((_N_ENT_PAD, _EMB), jnp.float32),
    )(psum, pcnt[0].reshape(_N_ENT_PAD, 1))


def _intent_vec(row, sub):
    logits = jnp.sum(row * sub, axis=1, keepdims=True)
    m = jnp.max(logits, axis=0, keepdims=True)
    e = jnp.exp(logits - m)
    att = e / jnp.sum(e, axis=0, keepdims=True)
    return jnp.sum(att * sub, axis=0, keepdims=True) / sub.shape[0]


def _user_body(u_ref, im_ref, ent_ref, it_ref, r_ref, out_ref):
    it = it_ref[...]
    r = r_ref[...]
    parts = [
        _intent_vec(it[0:1], r),
        _intent_vec(it[1:2], r[0:4]),
        _intent_vec(it[2:3], r[4:8]),
        _intent_vec(it[3:4], r[8:12]),
        _intent_vec(it[4:5], r[12:16]),
    ]
    all_intent = jnp.concatenate(parts, axis=0)
    new_intent = (all_intent + it) * 0.5

    u = u_ref[...]
    score_ = jax.lax.dot_general(
        u, new_intent, (((1,), (1,)), ((), ())),
        preferred_element_type=jnp.float32)
    sm = jnp.max(score_, axis=1, keepdims=True)
    se = jnp.exp(score_ - sm)
    score = se / jnp.sum(se, axis=1, keepdims=True)

    wvec = jax.lax.dot_general(
        score, new_intent, (((1,), (0,)), ((), ())),
        preferred_element_type=jnp.float32)

    agg = jax.lax.dot_general(
        im_ref[...], ent_ref[...], (((1,), (0,)), ((), ())),
        preferred_element_type=jnp.float32)
    out_ref[...] = agg * (1.0 + wvec)


def _user_agg(user_emb, interact_mat, entity_emb, intent_emb, r_emb):
    n_users = user_emb.shape[0]
    blk = 512
    grid = n_users // blk
    return pl.pallas_call(
        _user_body,
        grid=(grid,),
        in_specs=[
            pl.BlockSpec((blk, _EMB), lambda i: (i, 0)),
            pl.BlockSpec((blk, _N_ENT), lambda i: (i, 0)),
            pl.BlockSpec((_N_ENT, _EMB), lambda i: (0, 0)),
            pl.BlockSpec((5, _EMB), lambda i: (0, 0)),
            pl.BlockSpec((16, _EMB), lambda i: (0, 0)),
        ],
        out_specs=pl.BlockSpec((blk, _EMB), lambda i: (i, 0)),
        out_shape=jax.ShapeDtypeStruct((n_users, _EMB), jnp.float32),
    )(user_emb, interact_mat, entity_emb, intent_emb, r_emb)


def kernel(entity_emb, user_emb, intent_emb, edge_index, edge_type, interact_mat, r_emb):
    npad = _EPT_PAD - _EPT
    head = edge_index[0].astype(jnp.int32)
    tail = edge_index[1].astype(jnp.int32)
    rel = (edge_type.astype(jnp.int32) - 1) & 15
    packed = (tail | (head << 14) | (rel << 28)).reshape(_NT, _EPT)
    packed = jnp.concatenate(
        [packed, jnp.full((_NT, npad), _N_ENT << 14, jnp.int32)], axis=1
    ).reshape(_NT, _NTOT, _CHUNK)

    eh = jnp.stack([entity_emb[:, :_HALF], entity_emb[:, _HALF:]])
    eh = eh[:, :, _PINV].astype(jnp.bfloat16)
    tbl = jax.lax.bitcast_convert_type(
        eh.reshape(2, _N_ENT, _HW, 2), jnp.int32)
    remb_halves = jnp.stack([r_emb[:, :_HALF], r_emb[:, _HALF:]])

    psum, pcnt = _edge_agg(packed, tbl, remb_halves)
    entity_agg = _finalize(psum, pcnt)[:_N_ENT]
    user_agg = _user_agg(user_emb, interact_mat, entity_emb, intent_emb, r_emb)
    return entity_agg, user_agg
